# Initial kernel scaffold; baseline (speedup 1.0000x reference)
import jax
import jax.numpy as jnp
from jax.experimental import pallas as pl


def kernel(x, edge_index, Wl1, bl1, Wr1, Wl2, bl2, Wr2):
    raise NotImplementedError("write your pallas kernel here")


# speedup vs baseline: 5.2055x; 5.2055x over previous
import functools

import jax
import jax.numpy as jnp
from jax import lax
from jax.experimental import pallas as pl
from jax.experimental.pallas import tpu as pltpu
from jax.experimental.pallas import tpu_sc as plsc

NC = 2
NS = 16
NW = NC * NS
CK = 80


def _fill_rows(ref, nrows, width, value):
  vec = jnp.full((16,), value, dtype=jnp.float32)

  def body(r, carry):
    for j in range(width // 16):
      ref[r, pl.ds(j * 16, 16)] = vec
    return carry

  lax.fori_loop(0, nrows, body, 0)


def _zero_spmem_slice(acc, zbuf, row0, nrows, zrows):
  nfull = nrows // zrows
  rem = nrows % zrows

  def body(i, carry):
    pltpu.sync_copy(zbuf, acc.at[pl.ds(row0 + i * zrows, zrows)])
    return carry

  lax.fori_loop(0, nfull, body, 0)
  if rem:
    pltpu.sync_copy(zbuf.at[pl.ds(0, rem)],
                    acc.at[pl.ds(row0 + nfull * zrows, rem)])


def _make_edge_agg(n, d, e):
  assert d % 128 == 0 and e % NW == 0
  epw = e // NW
  assert epw % CK == 0 and epw % 8 == 0
  nf = epw // CK
  rps = (n // NS) // 8 * 8
  rrem = n - NS * rps

  mesh = plsc.VectorSubcoreMesh(
      core_axis_name="c", subcore_axis_name="s", num_cores=NC, num_subcores=NS)

  @functools.partial(
      pl.kernel,
      out_type=jax.ShapeDtypeStruct((NC, n, d), jnp.float32),
      mesh=mesh,
      scratch_types=[
          pltpu.VMEM((CK,), jnp.int32),
          pltpu.VMEM((CK,), jnp.int32),
          pltpu.VMEM((CK, d), jnp.float32),
          pltpu.MemorySpace.VMEM_SHARED((n, d), jnp.float32),
          pltpu.SemaphoreType.DMA,
      ],
      name=f"edge_agg_d{d}")
  def body(table, src, dst, agg_out, srcb, dstb, rows, acc, sem):
    ci = lax.axis_index("c")
    si = lax.axis_index("s")
    wid = si * NC + ci

    _fill_rows(rows, CK, d, 0.0)
    _zero_spmem_slice(acc, rows, si * rps, rps, CK)
    if rrem:
      @pl.when(si == 0)
      def _():
        _zero_spmem_slice(acc, rows, NS * rps, rrem, CK)
    plsc.subcore_barrier()

    base = wid * epw

    def loop_body(i, carry):
      off = base + i * CK
      pltpu.sync_copy(src.at[pl.ds(off, CK)], srcb)
      pltpu.sync_copy(dst.at[pl.ds(off, CK)], dstb)
      pltpu.async_copy(table.at[srcb], rows, sem).wait()
      pltpu.sync_copy(rows, acc.at[dstb], add=True)
      return carry

    lax.fori_loop(0, nf, loop_body, 0)

    plsc.subcore_barrier()

    pltpu.sync_copy(acc.at[pl.ds(si * rps, rps)],
                    agg_out.at[ci, pl.ds(si * rps, rps)])
    if rrem:
      @pl.when(si == 0)
      def _():
        pltpu.sync_copy(acc.at[pl.ds(NS * rps, rrem)],
                        agg_out.at[ci, pl.ds(NS * rps, rrem)])

  return body


def _make_counts(n, e):
  npad = -(-n // 128) * 128
  epw = e // NW
  nf = epw // CK
  assert nf * CK == epw

  mesh = plsc.VectorSubcoreMesh(
      core_axis_name="c", subcore_axis_name="s", num_cores=NC, num_subcores=NS)

  @functools.partial(
      pl.kernel,
      out_type=jax.ShapeDtypeStruct((NW * npad,), jnp.float32),
      mesh=mesh,
      scratch_types=[
          pltpu.VMEM((CK,), jnp.int32),
          pltpu.VMEM((npad,), jnp.float32),
      ],
      compiler_params=pltpu.CompilerParams(needs_layout_passes=False),
      name="degree_counts")
  def body(dst, cnt_out, dstb, hist):
    ci = lax.axis_index("c")
    si = lax.axis_index("s")
    wid = si * NC + ci

    z16 = jnp.zeros((16,), dtype=jnp.float32)

    def zbody(r, carry):
      hist[pl.ds(r * 16, 16)] = z16
      return carry

    lax.fori_loop(0, npad // 16, zbody, 0)

    ones16 = jnp.ones((16,), dtype=jnp.float32)
    base = wid * epw

    def loop_body(i, carry):
      pltpu.sync_copy(dst.at[pl.ds(base + i * CK, CK)], dstb)
      for k in range(CK // 16):
        v = dstb[pl.ds(k * 16, 16)]
        plsc.addupdate_scatter(hist, [v], ones16)
      return carry

    lax.fori_loop(0, nf, loop_body, 0)

    pltpu.sync_copy(hist, cnt_out.at[pl.ds(wid * npad, npad)])

  return body, npad


def _cnt_reduce_tc(cntw, npad):
  def body(c_ref, o_ref):
    o_ref[...] = jnp.clip(jnp.sum(c_ref[...], axis=0, keepdims=True),
                          1.0, None)

  return pl.pallas_call(
      body,
      out_shape=jax.ShapeDtypeStruct((1, npad), jnp.float32),
      name="cnt_reduce",
  )(cntw)


def _matmul_tc(x, w, block_m=2000):
  m, k = x.shape
  nn = w.shape[1]
  assert m % block_m == 0
  grid = (m // block_m,)

  def body(x_ref, w_ref, o_ref):
    o_ref[...] = jnp.dot(x_ref[...], w_ref[...],
                         preferred_element_type=jnp.float32)

  return pl.pallas_call(
      body,
      grid=grid,
      in_specs=[
          pl.BlockSpec((block_m, k), lambda i: (i, 0)),
          pl.BlockSpec((k, nn), lambda i: (0, 0)),
      ],
      out_specs=pl.BlockSpec((block_m, nn), lambda i: (i, 0)),
      out_shape=jax.ShapeDtypeStruct((m, nn), jnp.float32),
      name="dense_mm",
  )(x, w)


def _mid_tc(aggp, cnt1, z1, bl1r, block_m=2000):
  n, d_hid = z1.shape
  grid = (n // block_m,)

  def body(a_ref, c_ref, z_ref, b_ref, h_ref):
    agg = a_ref[0] + a_ref[1]
    h_ref[...] = jnp.maximum(
        agg / c_ref[...] + b_ref[...] + z_ref[...], 0.0)

  return pl.pallas_call(
      body,
      grid=grid,
      in_specs=[
          pl.BlockSpec((NC, block_m, d_hid), lambda i: (0, i, 0)),
          pl.BlockSpec((block_m, 1), lambda i: (i, 0)),
          pl.BlockSpec((block_m, d_hid), lambda i: (i, 0)),
          pl.BlockSpec((1, d_hid), lambda i: (0, 0)),
      ],
      out_specs=pl.BlockSpec((block_m, d_hid), lambda i: (i, 0)),
      out_shape=jax.ShapeDtypeStruct((n, d_hid), jnp.float32),
      name="mid_relu",
  )(aggp, cnt1, z1, bl1r)


def _final_tc(agg2p, cnt1, h, wl2, wr2, bl2r, block_m=2000):
  n, d_hid = h.shape
  d_out = wl2.shape[1]
  grid = (n // block_m,)

  def body(a_ref, c_ref, h_ref, wl_ref, wr_ref, b_ref, o_ref):
    m = (a_ref[0] + a_ref[1]) / c_ref[...]
    z = (jnp.dot(m, wl_ref[...], preferred_element_type=jnp.float32)
         + jnp.dot(h_ref[...], wr_ref[...], preferred_element_type=jnp.float32)
         + b_ref[...])
    o_ref[...] = jax.nn.sigmoid(z)

  return pl.pallas_call(
      body,
      grid=grid,
      in_specs=[
          pl.BlockSpec((NC, block_m, d_hid), lambda i: (0, i, 0)),
          pl.BlockSpec((block_m, 1), lambda i: (i, 0)),
          pl.BlockSpec((block_m, d_hid), lambda i: (i, 0)),
          pl.BlockSpec((d_hid, d_out), lambda i: (0, 0)),
          pl.BlockSpec((d_hid, d_out), lambda i: (0, 0)),
          pl.BlockSpec((1, d_out), lambda i: (0, 0)),
      ],
      out_specs=pl.BlockSpec((block_m, d_out), lambda i: (i, 0)),
      out_shape=jax.ShapeDtypeStruct((n, d_out), jnp.float32),
      name="final_mm_sigmoid",
  )(agg2p, cnt1, h, wl2, wr2, bl2r)


@jax.jit
def kernel(x, edge_index, Wl1, bl1, Wr1, Wl2, bl2, Wr2):
  n, d_in = x.shape
  e = edge_index.shape[1]
  d_hid = Wl1.shape[1]
  src = edge_index[0]
  dst = edge_index[1]

  xw = _matmul_tc(x, jnp.concatenate([Wl1, Wr1], axis=1))
  xwl1 = xw[:, :d_hid]
  xwr1 = xw[:, d_hid:]

  agg1p = _make_edge_agg(n, d_hid, e)(xwl1, src, dst)
  cnt_fn, npad = _make_counts(n, e)
  cntw = cnt_fn(dst).reshape(NW, npad)
  cnt1 = _cnt_reduce_tc(cntw, npad).reshape(npad)[:n].reshape(n, 1)

  h = _mid_tc(agg1p, cnt1, xwr1, bl1.reshape(1, -1))

  agg2p = _make_edge_agg(n, d_hid, e)(h, src, dst)

  return _final_tc(agg2p, cnt1, h, Wl2, Wr2, bl2.reshape(1, -1))


# speedup vs baseline: 10.5246x; 2.0218x over previous
import functools

import jax
import jax.numpy as jnp
from jax import lax
from jax.experimental import pallas as pl
from jax.experimental.pallas import tpu as pltpu
from jax.experimental.pallas import tpu_sc as plsc

NC = 2
NS = 16
NW = NC * NS
CK = 80


def _fill_rows(ref, nrows, width, value):
  vec = jnp.full((16,), value, dtype=jnp.float32)

  def body(r, carry):
    for j in range(width // 16):
      ref[r, pl.ds(j * 16, 16)] = vec
    return carry

  lax.fori_loop(0, nrows, body, 0)


def _zero_spmem_slice(acc, zbuf, row0, nrows, zrows):
  nfull = nrows // zrows
  rem = nrows % zrows

  def body(i, carry):
    pltpu.sync_copy(zbuf, acc.at[pl.ds(row0 + i * zrows, zrows)])
    return carry

  lax.fori_loop(0, nfull, body, 0)
  if rem:
    pltpu.sync_copy(zbuf.at[pl.ds(0, rem)],
                    acc.at[pl.ds(row0 + nfull * zrows, rem)])


def _make_edge_agg(n, d, nch, trash):
  assert d % 128 == 0
  GC = 8
  assert nch % GC == 0
  ng = nch // GC
  nacc = n + trash
  rps = (n // NS) // 8 * 8
  rrem = n - NS * rps

  mesh = plsc.VectorSubcoreMesh(
      core_axis_name="c", subcore_axis_name="s", num_cores=NC, num_subcores=NS)

  @functools.partial(
      pl.kernel,
      out_type=jax.ShapeDtypeStruct((NC, n, d), jnp.float32),
      mesh=mesh,
      scratch_types=[
          pltpu.VMEM((2, GC, CK), jnp.int32),
          pltpu.VMEM((2, GC, CK), jnp.int32),
          pltpu.VMEM((2, CK, d), jnp.float32),
          pltpu.MemorySpace.VMEM_SHARED((nacc, d), jnp.float32),
          [pltpu.SemaphoreType.DMA] * 2,
          [pltpu.SemaphoreType.DMA] * 2,
          [pltpu.SemaphoreType.DMA] * 2,
      ],
      name=f"edge_agg_d{d}")
  def body(table, src3, dst3, agg_out, sbuf, dbuf, rows, acc,
           isem, gsem, ssem):
    ci = lax.axis_index("c")
    si = lax.axis_index("s")
    wid = si * NC + ci

    _fill_rows(rows.at[0], CK, d, 0.0)
    _zero_spmem_slice(acc, rows.at[0], si * rps, rps, CK)
    if rrem:
      @pl.when(si == 0)
      def _():
        _zero_spmem_slice(acc, rows.at[0], NS * rps, rrem, CK)
    if trash:
      @pl.when(si == 1)
      def _():
        _zero_spmem_slice(acc, rows.at[0], n, trash, CK)
    plsc.subcore_barrier()

    idesc = {}
    gdesc = {}
    sdesc = {}

    def issue_idx(g):
      b = g % 2
      idesc[g] = (
          pltpu.async_copy(src3.at[wid, pl.ds(g * GC, GC)], sbuf.at[b],
                           isem[b]),
          pltpu.async_copy(dst3.at[wid, pl.ds(g * GC, GC)], dbuf.at[b],
                           isem[b]),
      )

    def issue_scatter(i):
      g, j = divmod(i, GC)
      sdesc[i] = pltpu.async_copy(
          rows.at[i % 2], acc.at[dbuf.at[g % 2, j]], ssem[i % 2], add=True)

    issue_idx(0)
    for g in range(ng):
      b = g % 2
      for dsc in idesc.pop(g):
        dsc.wait()
      for j in range(GC):
        i = g * GC + j
        p = i % 2
        if i >= 2:
          sdesc.pop(i - 2).wait()
        if j == 1 and g + 1 < ng:
          issue_idx(g + 1)
        gdesc[i] = pltpu.async_copy(table.at[sbuf.at[b, j]], rows.at[p],
                                    gsem[p])
        if i >= 1:
          gdesc.pop(i - 1).wait()
          issue_scatter(i - 1)
    last = nch - 1
    gdesc.pop(last).wait()
    issue_scatter(last)
    sdesc.pop(last - 1).wait()
    sdesc.pop(last).wait()

    plsc.subcore_barrier()

    pltpu.sync_copy(acc.at[pl.ds(si * rps, rps)],
                    agg_out.at[ci, pl.ds(si * rps, rps)])
    if rrem:
      @pl.when(si == 0)
      def _():
        pltpu.sync_copy(acc.at[pl.ds(NS * rps, rrem)],
                        agg_out.at[ci, pl.ds(NS * rps, rrem)])

  return body


def _make_counts(n, e, trash):
  npad = -(-(n + trash) // 128) * 128
  epw = e // NW
  nf = epw // CK
  assert nf * CK == epw

  mesh = plsc.VectorSubcoreMesh(
      core_axis_name="c", subcore_axis_name="s", num_cores=NC, num_subcores=NS)

  @functools.partial(
      pl.kernel,
      out_type=jax.ShapeDtypeStruct((NW * npad,), jnp.float32),
      mesh=mesh,
      scratch_types=[
          pltpu.VMEM((CK,), jnp.int32),
          pltpu.VMEM((npad,), jnp.float32),
      ],
      compiler_params=pltpu.CompilerParams(needs_layout_passes=False),
      name="degree_counts")
  def body(dst, cnt_out, dstb, hist):
    ci = lax.axis_index("c")
    si = lax.axis_index("s")
    wid = si * NC + ci

    z16 = jnp.zeros((16,), dtype=jnp.float32)

    def zbody(r, carry):
      hist[pl.ds(r * 16, 16)] = z16
      return carry

    lax.fori_loop(0, npad // 16, zbody, 0)

    ones16 = jnp.ones((16,), dtype=jnp.float32)
    base = wid * epw

    def loop_body(i, carry):
      pltpu.sync_copy(dst.at[pl.ds(base + i * CK, CK)], dstb)
      for k in range(CK // 16):
        v = dstb[pl.ds(k * 16, 16)]
        plsc.addupdate_scatter(hist, [v], ones16)
      return carry

    lax.fori_loop(0, nf, loop_body, 0)

    pltpu.sync_copy(hist, cnt_out.at[pl.ds(wid * npad, npad)])

  return body, npad


def _cnt_reduce_tc(cntw, npad):
  def body(c_ref, o_ref):
    o_ref[...] = jnp.clip(jnp.sum(c_ref[...], axis=0, keepdims=True),
                          1.0, None)

  return pl.pallas_call(
      body,
      out_shape=jax.ShapeDtypeStruct((1, npad), jnp.float32),
      name="cnt_reduce",
  )(cntw)


def _matmul_tc(x, w, block_m=2000):
  m, k = x.shape
  nn = w.shape[1]
  assert m % block_m == 0
  grid = (m // block_m,)

  def body(x_ref, w_ref, o_ref):
    o_ref[...] = jnp.dot(x_ref[...], w_ref[...],
                         preferred_element_type=jnp.float32)

  return pl.pallas_call(
      body,
      grid=grid,
      in_specs=[
          pl.BlockSpec((block_m, k), lambda i: (i, 0)),
          pl.BlockSpec((k, nn), lambda i: (0, 0)),
      ],
      out_specs=pl.BlockSpec((block_m, nn), lambda i: (i, 0)),
      out_shape=jax.ShapeDtypeStruct((m, nn), jnp.float32),
      name="dense_mm",
  )(x, w)


def _mid_tc(aggp, cnt1, z1, bl1r, block_m=2000):
  n, d_hid = z1.shape
  grid = (n // block_m,)

  def body(a_ref, c_ref, z_ref, b_ref, h_ref):
    agg = a_ref[0] + a_ref[1]
    h_ref[...] = jnp.maximum(
        agg / c_ref[...] + b_ref[...] + z_ref[...], 0.0)

  return pl.pallas_call(
      body,
      grid=grid,
      in_specs=[
          pl.BlockSpec((NC, block_m, d_hid), lambda i: (0, i, 0)),
          pl.BlockSpec((block_m, 1), lambda i: (i, 0)),
          pl.BlockSpec((block_m, d_hid), lambda i: (i, 0)),
          pl.BlockSpec((1, d_hid), lambda i: (0, 0)),
      ],
      out_specs=pl.BlockSpec((block_m, d_hid), lambda i: (i, 0)),
      out_shape=jax.ShapeDtypeStruct((n, d_hid), jnp.float32),
      name="mid_relu",
  )(aggp, cnt1, z1, bl1r)


def _final_tc(agg2p, cnt1, h, wl2, wr2, bl2r, block_m=2000):
  n, d_hid = h.shape
  d_out = wl2.shape[1]
  grid = (n // block_m,)

  def body(a_ref, c_ref, h_ref, wl_ref, wr_ref, b_ref, o_ref):
    m = (a_ref[0] + a_ref[1]) / c_ref[...]
    z = (jnp.dot(m, wl_ref[...], preferred_element_type=jnp.float32)
         + jnp.dot(h_ref[...], wr_ref[...], preferred_element_type=jnp.float32)
         + b_ref[...])
    o_ref[...] = jax.nn.sigmoid(z)

  return pl.pallas_call(
      body,
      grid=grid,
      in_specs=[
          pl.BlockSpec((NC, block_m, d_hid), lambda i: (0, i, 0)),
          pl.BlockSpec((block_m, 1), lambda i: (i, 0)),
          pl.BlockSpec((block_m, d_hid), lambda i: (i, 0)),
          pl.BlockSpec((d_hid, d_out), lambda i: (0, 0)),
          pl.BlockSpec((d_hid, d_out), lambda i: (0, 0)),
          pl.BlockSpec((1, d_out), lambda i: (0, 0)),
      ],
      out_specs=pl.BlockSpec((block_m, d_out), lambda i: (i, 0)),
      out_shape=jax.ShapeDtypeStruct((n, d_out), jnp.float32),
      name="final_mm_sigmoid",
  )(agg2p, cnt1, h, wl2, wr2, bl2r)


@jax.jit
def kernel(x, edge_index, Wl1, bl1, Wr1, Wl2, bl2, Wr2):
  n, d_in = x.shape
  e = edge_index.shape[1]
  d_hid = Wl1.shape[1]
  src = edge_index[0]
  dst = edge_index[1]

  trash = 64
  chpw = -(-(e // NW) // (8 * CK)) * 8
  epad = NW * chpw * CK
  extra = epad - e
  if extra:
    pad_ids = jnp.arange(extra, dtype=jnp.int32)
    src = jnp.concatenate([src, pad_ids % n])
    dst = jnp.concatenate([dst, n + pad_ids % trash])
  src3 = src.reshape(NW, chpw, CK)
  dst3 = dst.reshape(NW, chpw, CK)

  xw = _matmul_tc(x, jnp.concatenate([Wl1, Wr1], axis=1))
  xwl1 = xw[:, :d_hid]
  xwr1 = xw[:, d_hid:]

  agg1p = _make_edge_agg(n, d_hid, chpw, trash)(xwl1, src3, dst3)
  cnt_fn, npad = _make_counts(n, epad, trash)
  cntw = cnt_fn(dst).reshape(NW, npad)
  cnt1 = _cnt_reduce_tc(cntw, npad).reshape(npad)[:n].reshape(n, 1)

  h = _mid_tc(agg1p, cnt1, xwr1, bl1.reshape(1, -1))

  agg2p = _make_edge_agg(n, d_hid, chpw, trash)(h, src3, dst3)

  return _final_tc(agg2p, cnt1, h, Wl2, Wr2, bl2.reshape(1, -1))
